# baseline (speedup 1.0000x reference)
import dataclasses
import functools

import jax
import jax.numpy as jnp
from jax import lax
from jax.experimental import pallas as pl
from jax.experimental.pallas import tpu as pltpu
from jax.experimental.pallas import tpu_sc as plsc

N_E = 1000000
N_R = 1000
DIM = 64
B = 16384

NC = 2
NS = 16
NW = NC * NS
B_PER_W = B // NW
G = 128
NCH = B_PER_W // G
LANES = 16
W2 = 2 * DIM


def _sc_score_kernel(hrow_hbm, rrow_hbm, trow_hbm, hbase_hbm, rbase_hbm,
                     tbase_hbm, e2_hbm, r2_hbm, out_hbm,
                     idx_h, idx_r, idx_t, h_v, r_v, t_v, out_v,
                     hb_s, rb_s, tb_s, sem0, sem1):
    wid = lax.axis_index("s") * NC + lax.axis_index("c")

    pltpu.sync_copy(hrow_hbm.at[wid], idx_h)
    pltpu.sync_copy(rrow_hbm.at[wid], idx_r)
    pltpu.sync_copy(trow_hbm.at[wid], idx_t)
    pltpu.sync_copy(hbase_hbm.at[wid], hb_s)
    pltpu.sync_copy(rbase_hbm.at[wid], rb_s)
    pltpu.sync_copy(tbase_hbm.at[wid], tb_s)

    sems = (sem0, sem1)

    def fire(j):
        db, sem = j % 2, sems[j % 2]
        return [
            pltpu.async_copy(e2_hbm.at[idx_h.at[j]], h_v.at[db], sem),
            pltpu.async_copy(r2_hbm.at[idx_r.at[j]], r_v.at[db], sem),
            pltpu.async_copy(e2_hbm.at[idx_t.at[j]], t_v.at[db], sem),
        ]

    lane = lax.iota(jnp.int32, LANES)
    pend = fire(0)
    for j in range(NCH):
        nxt = fire(j + 1) if j + 1 < NCH else []
        for c in pend:
            c.wait()
        pend = nxt
        db = j % 2

        @pl.loop(0, G, step=LANES)
        def _(i0, j=j, db=db):
            row0 = j * G + i0
            bhv = hb_s[pl.ds(row0, LANES)]
            brv = rb_s[pl.ds(row0, LANES)]
            btv = tb_s[pl.ds(row0, LANES)]
            outv = jnp.zeros((LANES,), jnp.float32)
            for i in range(LANES):
                bh = bhv[i]
                br = brv[i]
                bt = btv[i]
                acc = jnp.zeros((LANES,), jnp.float32)
                for c in range(DIM // LANES):
                    o = c * LANES
                    hv = h_v[db, i0 + i, pl.ds(bh + o, LANES)]
                    rv = r_v[db, i0 + i, pl.ds(br + o, LANES)]
                    tv = t_v[db, i0 + i, pl.ds(bt + o, LANES)]
                    acc = acc + jnp.abs(hv + rv - tv)
                outv = jnp.where(lane == i, 100.0 - jnp.sum(acc), outv)
            out_v[pl.ds(j * G + i0, LANES)] = outv

    pltpu.sync_copy(out_v, out_hbm.at[pl.ds(wid * B_PER_W, B_PER_W)])


TBLK = 1024
TGRID = -(-N_E // TBLK)
NE2 = TGRID * (TBLK // 2)


def _tc_transpose_kernel(et_ref, out_ref):
    x = et_ref[...]
    out_ref[:, 0:DIM] = x[:, 0:TBLK // 2].T
    out_ref[:, DIM:W2] = x[:, TBLK // 2:TBLK].T


def _packed_rows(E_table):
    et = E_table.T
    return pl.pallas_call(
        _tc_transpose_kernel,
        out_shape=jax.ShapeDtypeStruct((NE2, W2), jnp.float32),
        grid=(TGRID,),
        in_specs=[pl.BlockSpec((DIM, TBLK), lambda j: (0, j))],
        out_specs=pl.BlockSpec((TBLK // 2, W2), lambda j: (j, 0)),
        compiler_params=pltpu.CompilerParams(
            dimension_semantics=("parallel",)),
    )(et)


@jax.jit
def kernel(heads, rels, tails, E_table, R_table):
    heads = heads.astype(jnp.int32)
    rels = rels.astype(jnp.int32)
    tails = tails.astype(jnp.int32)
    e2 = _packed_rows(E_table)
    r2 = R_table.reshape(N_R // 2, W2)

    def e_rows_bases(idx):
        m = idx & (TBLK - 1)
        row = ((idx >> 10) << 9) + (m & (TBLK // 2 - 1))
        base = (m >> 9) << 6
        return (jnp.reshape(row, (NW, NCH, G)),
                jnp.reshape(base, (NW, B_PER_W)))

    def r_rows_bases(idx):
        return (jnp.reshape(idx >> 1, (NW, NCH, G)),
                jnp.reshape((idx & 1) << 6, (NW, B_PER_W)))

    hrow, hbase = e_rows_bases(heads)
    rrow, rbase = r_rows_bases(rels)
    trow, tbase = e_rows_bases(tails)

    cp = pltpu.CompilerParams()
    for fld, val in (("needs_layout_passes", False),):
        if fld in pltpu.CompilerParams.__dataclass_fields__:
            cp = dataclasses.replace(cp, **{fld: val})
    mesh = plsc.VectorSubcoreMesh(core_axis_name="c", subcore_axis_name="s")
    run = pl.kernel(
        _sc_score_kernel,
        out_type=jax.ShapeDtypeStruct((B,), jnp.float32),
        mesh=mesh,
        compiler_params=cp,
        scratch_types=[
            pltpu.VMEM((NCH, G), jnp.int32),
            pltpu.VMEM((NCH, G), jnp.int32),
            pltpu.VMEM((NCH, G), jnp.int32),
            pltpu.VMEM((2, G, W2), jnp.float32),
            pltpu.VMEM((2, G, W2), jnp.float32),
            pltpu.VMEM((2, G, W2), jnp.float32),
            pltpu.VMEM((B_PER_W,), jnp.float32),
            pltpu.VMEM((B_PER_W,), jnp.int32),
            pltpu.VMEM((B_PER_W,), jnp.int32),
            pltpu.VMEM((B_PER_W,), jnp.int32),
            pltpu.SemaphoreType.DMA,
            pltpu.SemaphoreType.DMA,
        ],
    )
    return run(hrow, rrow, trow, hbase, rbase, tbase, e2, r2)


# speedup vs baseline: 1.8332x; 1.8332x over previous
import dataclasses
import functools

import jax
import jax.numpy as jnp
from jax import lax
from jax.experimental import pallas as pl
from jax.experimental.pallas import tpu as pltpu
from jax.experimental.pallas import tpu_sc as plsc

N_E = 1000000
N_R = 1000
DIM = 64
B = 16384

NC = 2
NS = 16
NW = NC * NS
B_PER_W = B // NW
G = 128
NCH = B_PER_W // G
LANES = 16
W2 = 2 * DIM


def _sc_score_kernel(hrow_hbm, rrow_hbm, trow_hbm, hbase_hbm, rbase_hbm,
                     tbase_hbm, e2_hbm, r2_hbm, out_hbm,
                     idx_h, idx_r, idx_t, h_v, r_v, t_v, out_v,
                     hb_s, rb_s, tb_s, sem0, sem1):
    wid = lax.axis_index("s") * NC + lax.axis_index("c")

    pltpu.sync_copy(hrow_hbm.at[wid], idx_h)
    pltpu.sync_copy(rrow_hbm.at[wid], idx_r)
    pltpu.sync_copy(trow_hbm.at[wid], idx_t)
    pltpu.sync_copy(hbase_hbm.at[wid], hb_s)
    pltpu.sync_copy(rbase_hbm.at[wid], rb_s)
    pltpu.sync_copy(tbase_hbm.at[wid], tb_s)

    sems = (sem0, sem1)

    def fire(j):
        db, sem = j % 2, sems[j % 2]
        return [
            pltpu.async_copy(e2_hbm.at[idx_h.at[j]], h_v.at[db], sem),
            pltpu.async_copy(r2_hbm.at[idx_r.at[j]], r_v.at[db], sem),
            pltpu.async_copy(e2_hbm.at[idx_t.at[j]], t_v.at[db], sem),
        ]

    lane = lax.iota(jnp.int32, LANES)
    pend = fire(0)
    for j in range(NCH):
        nxt = fire(j + 1) if j + 1 < NCH else []
        for c in pend:
            c.wait()
        pend = nxt
        db = j % 2

        @pl.loop(0, G, step=LANES)
        def _(i0, j=j, db=db):
            row0 = j * G + i0
            bhv = hb_s[pl.ds(row0, LANES)]
            brv = rb_s[pl.ds(row0, LANES)]
            btv = tb_s[pl.ds(row0, LANES)]
            outv = jnp.zeros((LANES,), jnp.float32)
            for i in range(LANES):
                bh = bhv[i]
                br = brv[i]
                bt = btv[i]
                acc = jnp.zeros((LANES,), jnp.float32)
                for c in range(DIM // LANES):
                    o = c * LANES
                    hv = h_v[db, i0 + i, pl.ds(bh + o, LANES)]
                    rv = r_v[db, i0 + i, pl.ds(br + o, LANES)]
                    tv = t_v[db, i0 + i, pl.ds(bt + o, LANES)]
                    acc = acc + jnp.abs(hv + rv - tv)
                outv = jnp.where(lane == i, 100.0 - jnp.sum(acc), outv)
            out_v[pl.ds(j * G + i0, LANES)] = outv

    pltpu.sync_copy(out_v, out_hbm.at[pl.ds(wid * B_PER_W, B_PER_W)])


TBLK = 1024
HALF = TBLK // 2
NFULL = N_E // TBLK
PER_CORE = NFULL // 2
TAILC = N_E - NFULL * TBLK
NE2 = (NFULL + 1) * HALF


def _tc_transpose_body(in_v, out_v):
    x = in_v[...]
    out_v[...] = jnp.concatenate([x[:, 0:HALF].T, x[:, HALF:TBLK].T], axis=1)


def _packed_rows(E_table):
    et = E_table.T
    mesh = pltpu.create_tensorcore_mesh("tc")

    @functools.partial(
        pl.kernel,
        out_type=jax.ShapeDtypeStruct((NE2, W2), jnp.float32),
        mesh=mesh,
        scratch_types=[
            pltpu.VMEM((DIM, TAILC), jnp.float32),
            pltpu.VMEM((HALF, W2), jnp.float32),
            pltpu.SemaphoreType.DMA,
        ],
    )
    def tr(et_hbm, out_hbm, tail_in_v, tail_out_v, sem):
        cid = lax.axis_index("tc")
        base = cid * PER_CORE
        pltpu.emit_pipeline(
            _tc_transpose_body,
            grid=(PER_CORE,),
            in_specs=[pl.BlockSpec((DIM, TBLK), lambda j: (0, base + j))],
            out_specs=[pl.BlockSpec((HALF, W2), lambda j: (base + j, 0))],
        )(et_hbm, out_hbm)

        @pl.when(cid == 0)
        def _():
            pltpu.async_copy(
                et_hbm.at[:, pl.ds(NFULL * TBLK, TAILC)], tail_in_v, sem
            ).wait()
            x = tail_in_v[...]
            z_a = x[:, 0:HALF].T
            z_b = jnp.concatenate(
                [x[:, HALF:TAILC].T,
                 jnp.zeros((TBLK - TAILC, DIM), jnp.float32)], axis=0)
            tail_out_v[...] = jnp.concatenate([z_a, z_b], axis=1)
            pltpu.async_copy(
                tail_out_v, out_hbm.at[pl.ds(NFULL * HALF, HALF)], sem
            ).wait()

    return tr(et)


@jax.jit
def kernel(heads, rels, tails, E_table, R_table):
    heads = heads.astype(jnp.int32)
    rels = rels.astype(jnp.int32)
    tails = tails.astype(jnp.int32)
    e2 = _packed_rows(E_table)
    r2 = R_table.reshape(N_R // 2, W2)

    def e_rows_bases(idx):
        m = idx & (TBLK - 1)
        row = (idx // TBLK) * HALF + (m & (HALF - 1))
        base = (m // HALF) << 6
        return (jnp.reshape(row, (NW, NCH, G)),
                jnp.reshape(base, (NW, B_PER_W)))

    def r_rows_bases(idx):
        return (jnp.reshape(idx >> 1, (NW, NCH, G)),
                jnp.reshape((idx & 1) << 6, (NW, B_PER_W)))

    hrow, hbase = e_rows_bases(heads)
    rrow, rbase = r_rows_bases(rels)
    trow, tbase = e_rows_bases(tails)

    cp = pltpu.CompilerParams()
    for fld, val in (("needs_layout_passes", False),):
        if fld in pltpu.CompilerParams.__dataclass_fields__:
            cp = dataclasses.replace(cp, **{fld: val})
    mesh = plsc.VectorSubcoreMesh(core_axis_name="c", subcore_axis_name="s")
    run = pl.kernel(
        _sc_score_kernel,
        out_type=jax.ShapeDtypeStruct((B,), jnp.float32),
        mesh=mesh,
        compiler_params=cp,
        scratch_types=[
            pltpu.VMEM((NCH, G), jnp.int32),
            pltpu.VMEM((NCH, G), jnp.int32),
            pltpu.VMEM((NCH, G), jnp.int32),
            pltpu.VMEM((2, G, W2), jnp.float32),
            pltpu.VMEM((2, G, W2), jnp.float32),
            pltpu.VMEM((2, G, W2), jnp.float32),
            pltpu.VMEM((B_PER_W,), jnp.float32),
            pltpu.VMEM((B_PER_W,), jnp.int32),
            pltpu.VMEM((B_PER_W,), jnp.int32),
            pltpu.VMEM((B_PER_W,), jnp.int32),
            pltpu.SemaphoreType.DMA,
            pltpu.SemaphoreType.DMA,
        ],
    )
    return run(hrow, rrow, trow, hbase, rbase, tbase, e2, r2)


# speedup vs baseline: 3.1043x; 1.6934x over previous
import dataclasses
import functools

import jax
import jax.numpy as jnp
from jax import lax
from jax.experimental import pallas as pl
from jax.experimental.pallas import tpu as pltpu
from jax.experimental.pallas import tpu_sc as plsc

N_E = 1000000
N_R = 1000
DIM = 64
B = 16384

NC = 2
NS = 16
NW = NC * NS
LANES = 16
W2 = 2 * DIM

EBLK = 512
BLKS = 61
SHARD = BLKS * EBLK
TAIL0 = NW * SHARD
VT0 = TAIL0 + EBLK
MCAP = 4112
BCAP = 160
NHT = 2 * B + NW

B_PER_W = B // NW
G = 128
NCH = B_PER_W // G


def _compiler_params(tc_tiling):
    cp = pltpu.CompilerParams()
    for fld, val in (("needs_layout_passes", False),
                     ("use_tc_tiling_on_sc", tc_tiling)):
        if fld in pltpu.CompilerParams.__dataclass_fields__:
            cp = dataclasses.replace(cp, **{fld: val})
    return cp


def _mesh():
    return plsc.VectorSubcoreMesh(core_axis_name="c", subcore_axis_name="s")


def _phase1(heads_hbm, tails_hbm, et_hbm, etail_hbm, ht_hbm,
            sbuf, ment, mslot, bent, bslot, buf0, buf1, stage, vtail,
            sem0, sem1):
    wid = lax.axis_index("s") * NC + lax.axis_index("c")
    e0 = wid * SHARD
    e1 = jnp.where(wid == NW - 1, N_E, e0 + SHARD)
    trash = 2 * B + wid
    lane = lax.iota(jnp.int32, LANES)
    bufs = (buf0, buf1)
    sems = (sem0, sem1)

    def scan_one(src_hbm, slot_off, cnt0):
        pltpu.sync_copy(src_hbm, sbuf)

        def grp(g, cnt):
            ev = sbuf[pl.ds(g * LANES, LANES)]
            mask = (ev >= e0) & (ev < e1)
            plsc.store_compressed(ment.at[pl.ds(cnt, LANES)], ev, mask=mask)
            sv = slot_off + g * LANES + lane
            plsc.store_compressed(mslot.at[pl.ds(cnt, LANES)], sv, mask=mask)
            return cnt + plsc.all_reduce_population_count(mask)[0]

        return lax.fori_loop(0, B // LANES, grp, cnt0)

    cnt = scan_one(heads_hbm, 0, jnp.int32(0))
    cnt = scan_one(tails_hbm, B, cnt)
    ngrp = (cnt + LANES - 1) // LANES

    def fire(k, db):
        @pl.when(k <= jnp.where(wid == NW - 1, BLKS, BLKS - 1))
        def _():
            pltpu.async_copy(
                et_hbm.at[:, pl.ds(e0 + k * EBLK, EBLK)], bufs[db], sems[db])

    def drain(db):
        pltpu.make_async_copy(
            et_hbm.at[:, pl.ds(0, EBLK)], bufs[db], sems[db]).wait()

    def extract16(g2, m, ev_all, sv_all, loader):
        valid = g2 * LANES + lane < m
        sv = jnp.where(valid, sv_all, trash)
        for i in range(LANES):
            e_s = ev_all[i]

            @pl.when(g2 * LANES + i < m)
            def _(e_s=e_s, i=i):
                for c in range(DIM // LANES):
                    stage[i, pl.ds(c * LANES, LANES)] = loader(e_s, c)

        pltpu.sync_copy(stage, ht_hbm.at[sv])

    def build_and_extract(lo, hi, loader):
        def grp2(g2, m):
            ev = ment[pl.ds(g2 * LANES, LANES)]
            sv = mslot[pl.ds(g2 * LANES, LANES)]
            mask = (ev >= lo) & (ev < hi)
            plsc.store_compressed(bent.at[pl.ds(m, LANES)], ev, mask=mask)
            plsc.store_compressed(bslot.at[pl.ds(m, LANES)], sv, mask=mask)
            return m + plsc.all_reduce_population_count(mask)[0]

        m = lax.fori_loop(0, ngrp, grp2, jnp.int32(0))

        def egrp(g3, z):
            cev = bent[pl.ds(g3 * LANES, LANES)]
            csv = bslot[pl.ds(g3 * LANES, LANES)]
            extract16(g3, m, cev, csv, loader)
            return z

        lax.fori_loop(0, (m + LANES - 1) // LANES, egrp, jnp.int32(0))

    def process(bstart, db):
        buf = bufs[db]

        def blk_loader(e_s, c):
            f_vec = c * LANES + lane
            eloc = jnp.full((LANES,), e_s - bstart, jnp.int32)
            return plsc.load_gather(buf, [f_vec, eloc])

        build_and_extract(bstart, bstart + EBLK, blk_loader)

    fire(0, 0)
    fire(1, 1)

    @pl.loop(0, BLKS // 2)
    def _(i):
        k = 2 * i
        drain(0)
        process(e0 + k * EBLK, 0)
        fire(k + 2, 0)
        drain(1)
        process(e0 + (k + 1) * EBLK, 1)
        fire(k + 3, 1)

    drain(0)
    process(e0 + (BLKS - 1) * EBLK, 0)

    @pl.when(wid == NW - 1)
    def _():
        drain(1)
        process(jnp.int32(TAIL0), 1)
        pltpu.sync_copy(etail_hbm, vtail)

        def vt_loader(e_s, c):
            d = e_s - VT0
            row = jnp.full((LANES,), d >> 1, jnp.int32)
            col = (d & 1) * DIM + c * LANES + lane
            return plsc.load_gather(vtail, [row, col])

        build_and_extract(jnp.int32(VT0), jnp.int32(N_E), vt_loader)


def _phase2(rrow_hbm, rbase_hbm, ht_hbm, r2_hbm, out_hbm,
            idx_r, h_v, r_v, t_v, out_v, rb_v, sem0, sem1):
    wid = lax.axis_index("s") * NC + lax.axis_index("c")
    slot0 = wid * B_PER_W

    pltpu.sync_copy(rrow_hbm.at[wid], idx_r)
    pltpu.sync_copy(rbase_hbm.at[wid], rb_v)

    sems = (sem0, sem1)

    def fire(j):
        db, sem = j % 2, sems[j % 2]
        return [
            pltpu.async_copy(
                ht_hbm.at[pl.ds(slot0 + j * G, G)], h_v.at[db], sem),
            pltpu.async_copy(
                ht_hbm.at[pl.ds(B + slot0 + j * G, G)], t_v.at[db], sem),
            pltpu.async_copy(r2_hbm.at[idx_r.at[j]], r_v.at[db], sem),
        ]

    lane = lax.iota(jnp.int32, LANES)
    pend = fire(0)
    for j in range(NCH):
        nxt = fire(j + 1) if j + 1 < NCH else []
        for cp_ in pend:
            cp_.wait()
        pend = nxt
        db = j % 2

        @pl.loop(0, G, step=LANES)
        def _(i0, j=j, db=db):
            row0 = j * G + i0
            brv = rb_v[pl.ds(row0, LANES)]
            outv = jnp.zeros((LANES,), jnp.float32)
            for i in range(LANES):
                br = brv[i]
                acc = jnp.zeros((LANES,), jnp.float32)
                for c in range(DIM // LANES):
                    o = c * LANES
                    hv = h_v[db, i0 + i, pl.ds(o, LANES)]
                    rv = r_v[db, i0 + i, pl.ds(br + o, LANES)]
                    tv = t_v[db, i0 + i, pl.ds(o, LANES)]
                    acc = acc + jnp.abs(hv + rv - tv)
                outv = jnp.where(lane == i, 100.0 - jnp.sum(acc), outv)
            out_v[pl.ds(j * G + i0, LANES)] = outv

    pltpu.sync_copy(out_v, out_hbm.at[pl.ds(slot0, B_PER_W)])


@jax.jit
def kernel(heads, rels, tails, E_table, R_table):
    heads = heads.astype(jnp.int32)
    rels = rels.astype(jnp.int32)
    tails = tails.astype(jnp.int32)

    et = E_table.T
    etail = E_table[VT0:].reshape((N_E - VT0) // 2, W2)

    run1 = pl.kernel(
        _phase1,
        out_type=jax.ShapeDtypeStruct((NHT, W2), jnp.float32),
        mesh=_mesh(),
        compiler_params=_compiler_params(True),
        scratch_types=[
            pltpu.VMEM((B,), jnp.int32),
            pltpu.VMEM((MCAP,), jnp.int32),
            pltpu.VMEM((MCAP,), jnp.int32),
            pltpu.VMEM((BCAP,), jnp.int32),
            pltpu.VMEM((BCAP,), jnp.int32),
            pltpu.VMEM((DIM, EBLK), jnp.float32),
            pltpu.VMEM((DIM, EBLK), jnp.float32),
            pltpu.VMEM((LANES, W2), jnp.float32),
            pltpu.VMEM(((N_E - VT0) // 2, W2), jnp.float32),
            pltpu.SemaphoreType.DMA,
            pltpu.SemaphoreType.DMA,
        ],
    )
    ht = run1(heads, tails, et, etail)

    r2 = R_table.reshape(N_R // 2, W2)
    rrow = jnp.reshape(rels >> 1, (NW, NCH, G))
    rbase = jnp.reshape((rels & 1) << 6, (NW, B_PER_W))

    run2 = pl.kernel(
        _phase2,
        out_type=jax.ShapeDtypeStruct((B,), jnp.float32),
        mesh=_mesh(),
        compiler_params=_compiler_params(False),
        scratch_types=[
            pltpu.VMEM((NCH, G), jnp.int32),
            pltpu.VMEM((2, G, W2), jnp.float32),
            pltpu.VMEM((2, G, W2), jnp.float32),
            pltpu.VMEM((2, G, W2), jnp.float32),
            pltpu.VMEM((B_PER_W,), jnp.float32),
            pltpu.VMEM((B_PER_W,), jnp.int32),
            pltpu.SemaphoreType.DMA,
            pltpu.SemaphoreType.DMA,
        ],
    )
    return run2(rrow, rbase, ht, r2)


# speedup vs baseline: 3.1364x; 1.0103x over previous
import dataclasses
import functools

import jax
import jax.numpy as jnp
from jax import lax
from jax.experimental import pallas as pl
from jax.experimental.pallas import tpu as pltpu
from jax.experimental.pallas import tpu_sc as plsc

N_E = 1000000
N_R = 1000
DIM = 64
B = 16384

NC = 2
NS = 16
NW = NC * NS
LANES = 16
W2 = 2 * DIM

EBLK = 512
BLKS = 61
SHARD = BLKS * EBLK
TAIL0 = NW * SHARD
VT0 = TAIL0 + EBLK
MCAP = 4112
BCAP = 160
NHT = 2 * B + NW

B_PER_W = B // NW
G = 128
NCH = B_PER_W // G


def _compiler_params(tc_tiling):
    cp = pltpu.CompilerParams()
    for fld, val in (("needs_layout_passes", False),
                     ("use_tc_tiling_on_sc", tc_tiling)):
        if fld in pltpu.CompilerParams.__dataclass_fields__:
            cp = dataclasses.replace(cp, **{fld: val})
    return cp


def _mesh():
    return plsc.VectorSubcoreMesh(core_axis_name="c", subcore_axis_name="s")


def _phase1(heads_hbm, tails_hbm, et_hbm, etail_hbm, ht_hbm,
            sbuf, ment, mslot, bent, bslot, buf0, buf1, stage, vtail,
            sem0, sem1, sem2):
    wid = lax.axis_index("s") * NC + lax.axis_index("c")
    e0 = wid * SHARD
    e1 = jnp.where(wid == NW - 1, N_E, e0 + SHARD)
    trash = 2 * B + wid
    lane = lax.iota(jnp.int32, LANES)
    bufs = (buf0, buf1)
    sems = (sem0, sem1)

    def scan_one(src_hbm, slot_off, cnt0):
        pltpu.sync_copy(src_hbm, sbuf)

        def grp(g, cnt):
            ev = sbuf[pl.ds(g * LANES, LANES)]
            mask = (ev >= e0) & (ev < e1)
            plsc.store_compressed(ment.at[pl.ds(cnt, LANES)], ev, mask=mask)
            sv = slot_off + g * LANES + lane
            plsc.store_compressed(mslot.at[pl.ds(cnt, LANES)], sv, mask=mask)
            return cnt + plsc.all_reduce_population_count(mask)[0]

        return lax.fori_loop(0, B // LANES, grp, cnt0)

    cnt = scan_one(heads_hbm, 0, jnp.int32(0))
    cnt = scan_one(tails_hbm, B, cnt)
    ngrp = (cnt + LANES - 1) // LANES

    def fire(k, db):
        @pl.when(k <= jnp.where(wid == NW - 1, BLKS, BLKS - 1))
        def _():
            pltpu.async_copy(
                et_hbm.at[:, pl.ds(e0 + k * EBLK, EBLK)],
                bufs[db].at[:, pl.ds(0, EBLK)], sems[db])

    def drain(db):
        pltpu.make_async_copy(
            et_hbm.at[:, pl.ds(0, EBLK)],
            bufs[db].at[:, pl.ds(0, EBLK)], sems[db]).wait()

    def drain_scatter():
        pltpu.make_async_copy(ht_hbm.at[pl.ds(0, LANES)], stage, sem2).wait()

    def extract16(g2, m, ev_all, sv_all, loader):
        valid = g2 * LANES + lane < m
        sv = jnp.where(valid, sv_all, trash)
        drain_scatter()
        for i in range(LANES):
            e_s = ev_all[i]

            @pl.when(g2 * LANES + i < m)
            def _(e_s=e_s, i=i):
                for c in range(DIM // LANES):
                    stage[i, pl.ds(c * LANES, LANES)] = loader(e_s, c)

        pltpu.async_copy(stage, ht_hbm.at[sv], sem2)

    def build_and_extract(lo, hi, loader):
        def grp2(g2, m):
            ev = ment[pl.ds(g2 * LANES, LANES)]
            sv = mslot[pl.ds(g2 * LANES, LANES)]
            mask = (ev >= lo) & (ev < hi)
            plsc.store_compressed(bent.at[pl.ds(m, LANES)], ev, mask=mask)
            plsc.store_compressed(bslot.at[pl.ds(m, LANES)], sv, mask=mask)
            return m + plsc.all_reduce_population_count(mask)[0]

        m = lax.fori_loop(0, ngrp, grp2, jnp.int32(0))

        def egrp(g3, z):
            cev = bent[pl.ds(g3 * LANES, LANES)]
            csv = bslot[pl.ds(g3 * LANES, LANES)]
            extract16(g3, m, cev, csv, loader)
            return z

        lax.fori_loop(0, (m + LANES - 1) // LANES, egrp, jnp.int32(0))

    def process(bstart, db):
        buf = bufs[db]

        def blk_loader(e_s, c):
            f_vec = c * LANES + lane
            eloc = jnp.full((LANES,), e_s - bstart, jnp.int32)
            return plsc.load_gather(buf, [f_vec, eloc])

        build_and_extract(bstart, bstart + EBLK, blk_loader)

    pltpu.async_copy(stage, ht_hbm.at[jnp.full((LANES,), trash, jnp.int32)],
                     sem2)

    fire(0, 0)
    fire(1, 1)

    @pl.loop(0, BLKS // 2)
    def _(i):
        k = 2 * i
        drain(0)
        process(e0 + k * EBLK, 0)
        fire(k + 2, 0)
        drain(1)
        process(e0 + (k + 1) * EBLK, 1)
        fire(k + 3, 1)

    drain(0)
    process(e0 + (BLKS - 1) * EBLK, 0)

    @pl.when(wid == NW - 1)
    def _():
        drain(1)
        process(jnp.int32(TAIL0), 1)
        pltpu.sync_copy(etail_hbm, vtail)

        def vt_loader(e_s, c):
            d = e_s - VT0
            row = jnp.full((LANES,), d >> 1, jnp.int32)
            col = (d & 1) * DIM + c * LANES + lane
            return plsc.load_gather(vtail, [row, col])

        build_and_extract(jnp.int32(VT0), jnp.int32(N_E), vt_loader)

    drain_scatter()


def _phase2(rrow_hbm, rbase_hbm, ht_hbm, r2_hbm, out_hbm,
            idx_r, h_v, r_v, t_v, out_v, rb_v, sem0, sem1):
    wid = lax.axis_index("s") * NC + lax.axis_index("c")
    slot0 = wid * B_PER_W

    pltpu.sync_copy(rrow_hbm.at[wid], idx_r)
    pltpu.sync_copy(rbase_hbm.at[wid], rb_v)

    sems = (sem0, sem1)

    def fire(j):
        db, sem = j % 2, sems[j % 2]
        return [
            pltpu.async_copy(
                ht_hbm.at[pl.ds(slot0 + j * G, G)], h_v.at[db], sem),
            pltpu.async_copy(
                ht_hbm.at[pl.ds(B + slot0 + j * G, G)], t_v.at[db], sem),
            pltpu.async_copy(r2_hbm.at[idx_r.at[j]], r_v.at[db], sem),
        ]

    lane = lax.iota(jnp.int32, LANES)
    pend = fire(0)
    for j in range(NCH):
        nxt = fire(j + 1) if j + 1 < NCH else []
        for cp_ in pend:
            cp_.wait()
        pend = nxt
        db = j % 2

        @pl.loop(0, G, step=LANES)
        def _(i0, j=j, db=db):
            row0 = j * G + i0
            brv = rb_v[pl.ds(row0, LANES)]
            outv = jnp.zeros((LANES,), jnp.float32)
            for i in range(LANES):
                br = brv[i]
                acc = jnp.zeros((LANES,), jnp.float32)
                for c in range(DIM // LANES):
                    o = c * LANES
                    hv = h_v[db, i0 + i, pl.ds(o, LANES)]
                    rv = r_v[db, i0 + i, pl.ds(br + o, LANES)]
                    tv = t_v[db, i0 + i, pl.ds(o, LANES)]
                    acc = acc + jnp.abs(hv + rv - tv)
                outv = jnp.where(lane == i, 100.0 - jnp.sum(acc), outv)
            out_v[pl.ds(j * G + i0, LANES)] = outv

    pltpu.sync_copy(out_v, out_hbm.at[pl.ds(slot0, B_PER_W)])


@jax.jit
def kernel(heads, rels, tails, E_table, R_table):
    heads = heads.astype(jnp.int32)
    rels = rels.astype(jnp.int32)
    tails = tails.astype(jnp.int32)

    et = E_table.T
    etail = E_table[VT0:].reshape((N_E - VT0) // 2, W2)

    run1 = pl.kernel(
        _phase1,
        out_type=jax.ShapeDtypeStruct((NHT, W2), jnp.float32),
        mesh=_mesh(),
        compiler_params=_compiler_params(True),
        scratch_types=[
            pltpu.VMEM((B,), jnp.int32),
            pltpu.VMEM((MCAP,), jnp.int32),
            pltpu.VMEM((MCAP,), jnp.int32),
            pltpu.VMEM((BCAP,), jnp.int32),
            pltpu.VMEM((BCAP,), jnp.int32),
            pltpu.VMEM((DIM, EBLK + 1), jnp.float32),
            pltpu.VMEM((DIM, EBLK + 1), jnp.float32),
            pltpu.VMEM((LANES, W2), jnp.float32),
            pltpu.VMEM(((N_E - VT0) // 2, W2), jnp.float32),
            pltpu.SemaphoreType.DMA,
            pltpu.SemaphoreType.DMA,
            pltpu.SemaphoreType.DMA,
        ],
    )
    ht = run1(heads, tails, et, etail)

    r2 = R_table.reshape(N_R // 2, W2)
    rrow = jnp.reshape(rels >> 1, (NW, NCH, G))
    rbase = jnp.reshape((rels & 1) << 6, (NW, B_PER_W))

    run2 = pl.kernel(
        _phase2,
        out_type=jax.ShapeDtypeStruct((B,), jnp.float32),
        mesh=_mesh(),
        compiler_params=_compiler_params(False),
        scratch_types=[
            pltpu.VMEM((NCH, G), jnp.int32),
            pltpu.VMEM((2, G, W2), jnp.float32),
            pltpu.VMEM((2, G, W2), jnp.float32),
            pltpu.VMEM((2, G, W2), jnp.float32),
            pltpu.VMEM((B_PER_W,), jnp.float32),
            pltpu.VMEM((B_PER_W,), jnp.int32),
            pltpu.SemaphoreType.DMA,
            pltpu.SemaphoreType.DMA,
        ],
    )
    return run2(rrow, rbase, ht, r2)
